# Initial kernel scaffold; baseline (speedup 1.0000x reference)
import jax
import jax.numpy as jnp
from jax.experimental import pallas as pl


def kernel(input):
    raise NotImplementedError("write your pallas kernel here")


# speedup vs baseline: 3.3012x; 3.3012x over previous
import jax
import jax.numpy as jnp
from jax.experimental import pallas as pl

NUM_CH = 2048
ROWS = 4 * 4096
BLOCK_ROWS = 512


def _rev_body(x_ref, o_ref):
    row = jax.lax.broadcasted_iota(jnp.int32, (128, 128), 0)
    col = jax.lax.broadcasted_iota(jnp.int32, (128, 128), 1)
    j = (row + col == 127).astype(jnp.float32)
    x = x_ref[...]
    for g in range(NUM_CH // 128):
        src = x[:, (NUM_CH // 128 - 1 - g) * 128:(NUM_CH // 128 - g) * 128]
        o_ref[:, g * 128:(g + 1) * 128] = jax.lax.dot(
            src, j, precision=jax.lax.Precision.HIGHEST)


def kernel(input):
    x = input.reshape(ROWS, NUM_CH)
    out = pl.pallas_call(
        _rev_body,
        grid=(ROWS // BLOCK_ROWS,),
        in_specs=[pl.BlockSpec((BLOCK_ROWS, NUM_CH), lambda i: (i, 0))],
        out_specs=pl.BlockSpec((BLOCK_ROWS, NUM_CH), lambda i: (i, 0)),
        out_shape=jax.ShapeDtypeStruct((ROWS, NUM_CH), jnp.float32),
    )(x)
    return out.reshape(input.shape)


# speedup vs baseline: 4.2378x; 1.2837x over previous
import jax
import jax.numpy as jnp
from jax.experimental import pallas as pl

NUM_CH = 2048
ROWS = 4 * 4096
BLOCK_ROWS = 512


def _rev_body(x_ref, o_ref):
    row = jax.lax.broadcasted_iota(jnp.int32, (128, 128), 0)
    col = jax.lax.broadcasted_iota(jnp.int32, (128, 128), 1)
    j = (row + col == 127).astype(jnp.bfloat16)
    x = x_ref[...]
    hi = x.astype(jnp.bfloat16)
    r1 = x - hi.astype(jnp.float32)
    mid = r1.astype(jnp.bfloat16)
    lo = (r1 - mid.astype(jnp.float32)).astype(jnp.bfloat16)
    for g in range(NUM_CH // 128):
        sl = slice((NUM_CH // 128 - 1 - g) * 128, (NUM_CH // 128 - g) * 128)
        acc = jax.lax.dot(hi[:, sl], j, preferred_element_type=jnp.float32)
        acc += jax.lax.dot(mid[:, sl], j, preferred_element_type=jnp.float32)
        acc += jax.lax.dot(lo[:, sl], j, preferred_element_type=jnp.float32)
        o_ref[:, g * 128:(g + 1) * 128] = acc


def kernel(input):
    x = input.reshape(ROWS, NUM_CH)
    out = pl.pallas_call(
        _rev_body,
        grid=(ROWS // BLOCK_ROWS,),
        in_specs=[pl.BlockSpec((BLOCK_ROWS, NUM_CH), lambda i: (i, 0))],
        out_specs=pl.BlockSpec((BLOCK_ROWS, NUM_CH), lambda i: (i, 0)),
        out_shape=jax.ShapeDtypeStruct((ROWS, NUM_CH), jnp.float32),
    )(x)
    return out.reshape(input.shape)


# speedup vs baseline: 4.4446x; 1.0488x over previous
import jax
import jax.numpy as jnp
from jax.experimental import pallas as pl

NUM_CH = 2048
ROWS = 4 * 4096
BLOCK_ROWS = 1024


def _rev_body(x_ref, o_ref):
    row = jax.lax.broadcasted_iota(jnp.int32, (128, 128), 0)
    col = jax.lax.broadcasted_iota(jnp.int32, (128, 128), 1)
    j = (row + col == 127).astype(jnp.bfloat16)
    x = x_ref[...]
    hi = x.astype(jnp.bfloat16)
    r1 = x - hi.astype(jnp.float32)
    mid = r1.astype(jnp.bfloat16)
    lo = (r1 - mid.astype(jnp.float32)).astype(jnp.bfloat16)
    for g in range(NUM_CH // 128):
        sl = slice((NUM_CH // 128 - 1 - g) * 128, (NUM_CH // 128 - g) * 128)
        acc = jax.lax.dot(hi[:, sl], j, preferred_element_type=jnp.float32)
        acc += jax.lax.dot(mid[:, sl], j, preferred_element_type=jnp.float32)
        acc += jax.lax.dot(lo[:, sl], j, preferred_element_type=jnp.float32)
        o_ref[:, g * 128:(g + 1) * 128] = acc


def kernel(input):
    x = input.reshape(ROWS, NUM_CH)
    out = pl.pallas_call(
        _rev_body,
        grid=(ROWS // BLOCK_ROWS,),
        in_specs=[pl.BlockSpec((BLOCK_ROWS, NUM_CH), lambda i: (i, 0))],
        out_specs=pl.BlockSpec((BLOCK_ROWS, NUM_CH), lambda i: (i, 0)),
        out_shape=jax.ShapeDtypeStruct((ROWS, NUM_CH), jnp.float32),
    )(x)
    return out.reshape(input.shape)
